# baseline (speedup 1.0000x reference)
import jax
import jax.numpy as jnp
from jax.experimental import pallas as pl

_B, _Q, _D, _H, _HD, _M, _K = 32, 8, 1024, 16, 64, 32768, 32
_SCALE = 0.125
_R = 32
_CC = 64
_NC = _M // _CC
_DEPTH = 4


def _proj_kernel(a_ref, w_ref, o_ref):
    o_ref[...] = jnp.dot(a_ref[...], w_ref[...],
                         preferred_element_type=jnp.float32)


def _knn_kernel(q_ref, kt_ref, v_ref, o_ref):
    q = q_ref[0]
    s = jnp.dot(q, kt_ref[...],
                preferred_element_type=jnp.float32) * _SCALE
    neg = -jnp.inf
    s3 = s.reshape(_R, _NC, _CC)
    tc = jnp.max(s3, axis=2, keepdims=True)
    vs = [tc[:, :, 0]]
    for _ in range(_DEPTH - 1):
        tc = jnp.max(jnp.where(s3 < tc, s3, neg), axis=2, keepdims=True)
        vs.append(tc[:, :, 0])
    cand = jnp.concatenate(vs, axis=1)
    m1 = jnp.max(vs[0], axis=1, keepdims=True)
    t = m1
    for _ in range(_K - 1):
        t = jnp.max(jnp.where(cand < t, cand, neg), axis=1, keepdims=True)
    nc = jnp.sum(jnp.where(s >= t, 1.0, 0.0), axis=1, keepdims=True)

    def _exact(_):
        tt = m1
        for _ in range(_K - 1):
            tt = jnp.max(jnp.where(s < tt, s, neg), axis=1, keepdims=True)
        return tt

    t = jax.lax.cond(jnp.any(nc != float(_K)), _exact, lambda _: t, 0)
    w = jnp.where(s >= t, jnp.exp(s - m1), 0.0)
    denom = jnp.sum(w, axis=1, keepdims=True)
    ctx = jnp.dot(w, v_ref[...], preferred_element_type=jnp.float32)
    o_ref[0] = ctx / denom


def kernel(hidden_states, mem_keys, mem_values, W_q, W_o):
    hs = hidden_states.reshape(_B * _Q, _D)
    qp = pl.pallas_call(
        _proj_kernel,
        out_shape=jax.ShapeDtypeStruct((_B * _Q, _D), jnp.float32),
    )(hs, W_q)
    q4 = qp.reshape(_B, _Q, _H, _HD).transpose(0, 2, 1, 3).reshape(
        _B, _H * _Q, _HD)
    kt = mem_keys.T
    nblk = (_B * _H * _Q) // _R
    q4 = q4.reshape(nblk, _R, _HD)
    ctx = pl.pallas_call(
        _knn_kernel,
        grid=(nblk,),
        in_specs=[
            pl.BlockSpec((1, _R, _HD), lambda i: (i, 0, 0)),
            pl.BlockSpec((_HD, _M), lambda i: (0, 0)),
            pl.BlockSpec((_M, _HD), lambda i: (0, 0)),
        ],
        out_specs=pl.BlockSpec((1, _R, _HD), lambda i: (i, 0, 0)),
        out_shape=jax.ShapeDtypeStruct((nblk, _R, _HD), jnp.float32),
    )(q4, kt, mem_values)
    ctx2 = ctx.reshape(_B, _H, _Q, _HD).transpose(0, 2, 1, 3).reshape(
        _B * _Q, _D)
    out = pl.pallas_call(
        _proj_kernel,
        out_shape=jax.ShapeDtypeStruct((_B * _Q, _D), jnp.float32),
    )(ctx2, W_o)
    return out.reshape(_B, _Q, _D)


# speedup vs baseline: 1.3825x; 1.3825x over previous
import jax
import jax.numpy as jnp
from jax.experimental import pallas as pl

_B, _Q, _D, _H, _HD, _M, _K = 32, 8, 1024, 16, 64, 32768, 32
_SCALE = 0.125
_R = 32
_CC = 128
_NC = _M // _CC
_DEPTH = 5


def _proj_kernel(a_ref, w_ref, o_ref):
    o_ref[...] = jnp.dot(a_ref[...], w_ref[...],
                         preferred_element_type=jnp.float32)


def _knn_kernel(q_ref, kt_ref, v_ref, o_ref):
    q = q_ref[0]
    s = jnp.dot(q, kt_ref[...],
                preferred_element_type=jnp.float32) * _SCALE
    neg = -jnp.inf
    s3 = s.reshape(_R, _NC, _CC)
    tc = jnp.max(s3, axis=2, keepdims=True)
    vs = [tc[:, :, 0]]
    for _ in range(_DEPTH - 1):
        tc = jnp.max(jnp.where(s3 < tc, s3, neg), axis=2, keepdims=True)
        vs.append(tc[:, :, 0])
    cand = jnp.concatenate(vs, axis=1)
    m1 = jnp.max(vs[0], axis=1, keepdims=True)
    t = m1
    for _ in range(_K - 1):
        t = jnp.max(jnp.where(cand < t, cand, neg), axis=1, keepdims=True)
    nc = jnp.sum(jnp.where(s >= t, 1.0, 0.0), axis=1, keepdims=True)

    def _exact(_):
        tt = m1
        for _ in range(_K - 1):
            tt = jnp.max(jnp.where(s < tt, s, neg), axis=1, keepdims=True)
        return tt

    t = jax.lax.cond(jnp.any(nc != float(_K)), _exact, lambda _: t, 0)
    w = jnp.where(s >= t, jnp.exp(s - m1), 0.0)
    denom = jnp.sum(w, axis=1, keepdims=True)
    ctx = jnp.dot(w, v_ref[...], preferred_element_type=jnp.float32)
    o_ref[0] = ctx / denom


def kernel(hidden_states, mem_keys, mem_values, W_q, W_o):
    hs = hidden_states.reshape(_B * _Q, _D)
    qp = pl.pallas_call(
        _proj_kernel,
        out_shape=jax.ShapeDtypeStruct((_B * _Q, _D), jnp.float32),
    )(hs, W_q)
    q4 = qp.reshape(_B, _Q, _H, _HD).transpose(0, 2, 1, 3).reshape(
        _B, _H * _Q, _HD)
    kt = mem_keys.T
    nblk = (_B * _H * _Q) // _R
    q4 = q4.reshape(nblk, _R, _HD)
    ctx = pl.pallas_call(
        _knn_kernel,
        grid=(nblk,),
        in_specs=[
            pl.BlockSpec((1, _R, _HD), lambda i: (i, 0, 0)),
            pl.BlockSpec((_HD, _M), lambda i: (0, 0)),
            pl.BlockSpec((_M, _HD), lambda i: (0, 0)),
        ],
        out_specs=pl.BlockSpec((1, _R, _HD), lambda i: (i, 0, 0)),
        out_shape=jax.ShapeDtypeStruct((nblk, _R, _HD), jnp.float32),
    )(q4, kt, mem_values)
    ctx2 = ctx.reshape(_B, _H, _Q, _HD).transpose(0, 2, 1, 3).reshape(
        _B * _Q, _D)
    out = pl.pallas_call(
        _proj_kernel,
        out_shape=jax.ShapeDtypeStruct((_B * _Q, _D), jnp.float32),
    )(ctx2, W_o)
    return out.reshape(_B, _Q, _D)


# speedup vs baseline: 1.5336x; 1.1093x over previous
import jax
import jax.numpy as jnp
from jax.experimental import pallas as pl

_B, _Q, _D, _H, _HD, _M, _K = 32, 8, 1024, 16, 64, 32768, 32
_SCALE = 0.125
_R = 32
_CC = 128
_NC = _M // _CC
_DEPTH = 4


def _proj_kernel(a_ref, w_ref, o_ref):
    o_ref[...] = jnp.dot(a_ref[...], w_ref[...],
                         preferred_element_type=jnp.float32)


def _knn_kernel(q_ref, kt_ref, v_ref, o_ref):
    q = q_ref[0]
    s = jnp.dot(q, kt_ref[...],
                preferred_element_type=jnp.float32) * _SCALE
    neg = -jnp.inf
    s3 = s.reshape(_R, _NC, _CC)
    tc = jnp.max(s3, axis=2, keepdims=True)
    vs = [tc[:, :, 0]]
    for _ in range(_DEPTH - 1):
        tc = jnp.max(jnp.where(s3 < tc, s3, neg), axis=2, keepdims=True)
        vs.append(tc[:, :, 0])
    cand = jnp.concatenate(vs, axis=1)
    m1 = jnp.max(vs[0], axis=1, keepdims=True)
    t = m1
    for _ in range(_K - 1):
        t = jnp.max(jnp.where(cand < t, cand, neg), axis=1, keepdims=True)
    nc = jnp.sum(jnp.where(s >= t, 1.0, 0.0), axis=1, keepdims=True)

    def _exact(_):
        tt = m1
        for _ in range(_K - 1):
            tt = jnp.max(jnp.where(s < tt, s, neg), axis=1, keepdims=True)
        return tt

    t = jax.lax.cond(jnp.any(nc != float(_K)), _exact, lambda _: t, 0)
    w = jnp.where(s >= t, jnp.exp(s - m1), 0.0)
    denom = jnp.sum(w, axis=1, keepdims=True)
    ctx = jnp.dot(w, v_ref[...], preferred_element_type=jnp.float32)
    o_ref[0] = ctx / denom


def kernel(hidden_states, mem_keys, mem_values, W_q, W_o):
    hs = hidden_states.reshape(_B * _Q, _D)
    qp = pl.pallas_call(
        _proj_kernel,
        out_shape=jax.ShapeDtypeStruct((_B * _Q, _D), jnp.float32),
    )(hs, W_q)
    q4 = qp.reshape(_B, _Q, _H, _HD).transpose(0, 2, 1, 3).reshape(
        _B, _H * _Q, _HD)
    kt = mem_keys.T
    nblk = (_B * _H * _Q) // _R
    q4 = q4.reshape(nblk, _R, _HD)
    ctx = pl.pallas_call(
        _knn_kernel,
        grid=(nblk,),
        in_specs=[
            pl.BlockSpec((1, _R, _HD), lambda i: (i, 0, 0)),
            pl.BlockSpec((_HD, _M), lambda i: (0, 0)),
            pl.BlockSpec((_M, _HD), lambda i: (0, 0)),
        ],
        out_specs=pl.BlockSpec((1, _R, _HD), lambda i: (i, 0, 0)),
        out_shape=jax.ShapeDtypeStruct((nblk, _R, _HD), jnp.float32),
    )(q4, kt, mem_values)
    ctx2 = ctx.reshape(_B, _H, _Q, _HD).transpose(0, 2, 1, 3).reshape(
        _B * _Q, _D)
    out = pl.pallas_call(
        _proj_kernel,
        out_shape=jax.ShapeDtypeStruct((_B * _Q, _D), jnp.float32),
    )(ctx2, W_o)
    return out.reshape(_B, _Q, _D)
